# baseline (speedup 1.0000x reference)
import functools

import jax
import jax.numpy as jnp
from jax import lax
from jax.experimental import pallas as pl
from jax.experimental.pallas import tpu as pltpu
from jax.experimental.pallas import tpu_sc as plsc

NODE_IN = 256
EDGE_IN = 16
LEAK = 0.1
N_NODES = 10000
N_EDGES = 160000
H1 = 64

_HIGH = lax.Precision.HIGHEST


def _lrelu(v):
    return jnp.where(v >= 0, v, LEAK * v)


_E_STEPS = 5
_E_BLK = N_EDGES // _E_STEPS


def _estats_body(e_ref, ge_ref, be_ref, se_ref, te_ref, acce_ref):
    i = pl.program_id(0)

    @pl.when(i == 0)
    def _init():
        acce_ref[...] = jnp.zeros_like(acce_ref)

    eb = e_ref[...]
    acce_ref[0:1, :] += jnp.sum(eb, axis=0, keepdims=True)
    acce_ref[1:2, :] += jnp.sum(eb * eb, axis=0, keepdims=True)

    @pl.when(i == _E_STEPS - 1)
    def _finish():
        mu_e = acce_ref[0:1, :] / N_EDGES
        var_e = acce_ref[1:2, :] / N_EDGES - mu_e * mu_e
        se = ge_ref[...] * lax.rsqrt(var_e + 1e-5)
        se_ref[...] = se
        te_ref[...] = be_ref[...] - mu_e * se


def _run_estats(e, ge, be):
    const = lambda *_: (0, 0)
    return pl.pallas_call(
        _estats_body,
        grid=(_E_STEPS,),
        in_specs=[
            pl.BlockSpec((_E_BLK, EDGE_IN), lambda i: (i, 0)),
            pl.BlockSpec((1, EDGE_IN), const),
            pl.BlockSpec((1, EDGE_IN), const),
        ],
        out_specs=(
            pl.BlockSpec((1, EDGE_IN), const),
            pl.BlockSpec((1, EDGE_IN), const),
        ),
        out_shape=(
            jax.ShapeDtypeStruct((1, EDGE_IN), jnp.float32),
            jax.ShapeDtypeStruct((1, EDGE_IN), jnp.float32),
        ),
        scratch_shapes=[pltpu.VMEM((2, EDGE_IN), jnp.float32)],
    )(e, ge, be)


def _proj_body(x_ref, gn_ref, bn_ref, w1r_ref, w1c_ref, b1_ref,
               t_ref, t2_ref, b1e_ref):
    x = x_ref[...]
    mu = jnp.mean(x, axis=0, keepdims=True)
    xm = x - mu
    var = jnp.mean(xm * xm, axis=0, keepdims=True)
    sx = gn_ref[...] * lax.rsqrt(var + 1e-5)
    tx = bn_ref[...] - mu * sx
    xbn = x * sx + tx
    w1r = w1r_ref[...]
    w1c = w1c_ref[...]
    xr = jnp.dot(xbn, w1r, preferred_element_type=jnp.float32)
    xc = jnp.dot(xbn, w1c, preferred_element_type=jnp.float32)
    t_ref[...] = xr
    t2_ref[...] = xc
    b1e_ref[...] = (b1_ref[...]
                    + jnp.dot(tx, w1r, preferred_element_type=jnp.float32)
                    + jnp.dot(tx, w1c, preferred_element_type=jnp.float32))


def _run_proj(x, gn, bn, w1r_t, w1c_t, b1):
    return pl.pallas_call(
        _proj_body,
        out_shape=(
            jax.ShapeDtypeStruct((N_NODES, H1), jnp.float32),
            jax.ShapeDtypeStruct((N_NODES, H1), jnp.float32),
            jax.ShapeDtypeStruct((1, H1), jnp.float32),
        ),
    )(x, gn, bn, w1r_t, w1c_t, b1)


_NW = 32
_GC = 128


def _make_gather_body(epw, nfull, rem):
    def _gather_body(tab_hbm, tab2_hbm, row_hbm, col_hbm, gsum_hbm,
                     idx_row, idx_col, br0, br1, bc0, bc1, ob0, ob1,
                     sem_r0, sem_r1, sem_c0, sem_c1, sem_o0, sem_o1):
        wid = lax.axis_index("s") * 2 + lax.axis_index("c")
        base0 = wid * epw
        pltpu.sync_copy(row_hbm.at[pl.ds(base0, epw)], idx_row)
        pltpu.sync_copy(col_hbm.at[pl.ds(base0, epw)], idx_col)

        brs = (br0, br1)
        bcs = (bc0, bc1)
        obs = (ob0, ob1)
        sem_rs = (sem_r0, sem_r1)
        sem_cs = (sem_c0, sem_c1)
        sem_os = (sem_o0, sem_o1)

        def fire(j, b):
            sl = pl.ds(j * _GC, _GC)
            pltpu.async_copy(tab_hbm.at[idx_row.at[sl]], brs[b], sem_rs[b])
            pltpu.async_copy(tab2_hbm.at[idx_col.at[sl]], bcs[b], sem_cs[b])

        def wait_gather(b):
            sl = pl.ds(0, _GC)
            pltpu.make_async_copy(tab_hbm.at[idx_row.at[sl]], brs[b],
                                  sem_rs[b]).wait()
            pltpu.make_async_copy(tab2_hbm.at[idx_col.at[sl]], bcs[b],
                                  sem_cs[b]).wait()

        pbase0 = wid * (epw // 2)

        def wait_out(b):
            pltpu.make_async_copy(obs[b],
                                  gsum_hbm.at[pl.ds(pbase0, _GC // 2)],
                                  sem_os[b]).wait()

        def process(j, b):
            def row_add(p, carry):
                for half in range(2):
                    r = 2 * p + half
                    for q in range(H1 // 16):
                        obs[b][p, pl.ds(half * H1 + q * 16, 16)] = (
                            brs[b][r, pl.ds(q * 16, 16)]
                            + bcs[b][r, pl.ds(q * 16, 16)])
                return carry
            lax.fori_loop(0, _GC // 2, row_add, 0)
            pltpu.async_copy(obs[b],
                             gsum_hbm.at[pl.ds(pbase0 + j * (_GC // 2),
                                               _GC // 2)],
                             sem_os[b])

        fire(0, 0)
        fire(1, 1)

        def outer(j0, carry):
            for b in range(2):
                j = 2 * j0 + b
                wait_gather(b)

                @pl.when(j >= 2)
                def _():
                    wait_out(b)
                process(j, b)

                @pl.when(j + 2 < nfull)
                def _():
                    fire(j + 2, b)
            return carry

        lax.fori_loop(0, (nfull - 1) // 2, outer, 0)

        j_last = nfull - 1
        wait_gather(0)
        wait_out(0)
        process(j_last, 0)

        sl = pl.ds(nfull * _GC, rem)
        rem_r = br1.at[pl.ds(0, rem)]
        rem_c = bc1.at[pl.ds(0, rem)]
        wait_out(1)
        pltpu.async_copy(tab_hbm.at[idx_row.at[sl]], rem_r, sem_r1)
        pltpu.async_copy(tab2_hbm.at[idx_col.at[sl]], rem_c, sem_c1)
        pltpu.make_async_copy(tab_hbm.at[idx_row.at[sl]], rem_r,
                              sem_r1).wait()
        pltpu.make_async_copy(tab2_hbm.at[idx_col.at[sl]], rem_c,
                              sem_c1).wait()

        def rem_add(p, carry):
            for half in range(2):
                r = 2 * p + half
                for q in range(H1 // 16):
                    ob1[p, pl.ds(half * H1 + q * 16, 16)] = (
                        br1[r, pl.ds(q * 16, 16)]
                        + bc1[r, pl.ds(q * 16, 16)])
            return carry
        lax.fori_loop(0, rem // 2, rem_add, 0)
        pltpu.sync_copy(ob1.at[pl.ds(0, rem // 2)],
                        gsum_hbm.at[pl.ds(pbase0 + nfull * (_GC // 2),
                                          rem // 2)])
        wait_out(0)

    return _gather_body


def _run_gather(tab, tab2, row, col):
    n = row.shape[0]
    epw = n // _NW
    nfull = epw // _GC
    rem = epw - nfull * _GC
    assert n % (_NW * 8) == 0 and nfull % 2 == 1 and rem % 2 == 0 and rem > 0
    mesh = plsc.VectorSubcoreMesh(core_axis_name="c", subcore_axis_name="s",
                                  num_cores=2, num_subcores=16)
    fn = pl.kernel(
        _make_gather_body(epw, nfull, rem),
        out_type=jax.ShapeDtypeStruct((n // 2, 2 * H1), jnp.float32),
        mesh=mesh,
        compiler_params=pltpu.CompilerParams(use_tc_tiling_on_sc=False),
        scratch_types=[
            pltpu.VMEM((epw,), jnp.int32),
            pltpu.VMEM((epw,), jnp.int32),
            pltpu.VMEM((_GC, H1), jnp.float32),
            pltpu.VMEM((_GC, H1), jnp.float32),
            pltpu.VMEM((_GC, H1), jnp.float32),
            pltpu.VMEM((_GC, H1), jnp.float32),
            pltpu.VMEM((_GC // 2, 2 * H1), jnp.float32),
            pltpu.VMEM((_GC // 2, 2 * H1), jnp.float32),
            pltpu.SemaphoreType.DMA,
            pltpu.SemaphoreType.DMA,
            pltpu.SemaphoreType.DMA,
            pltpu.SemaphoreType.DMA,
            pltpu.SemaphoreType.DMA,
            pltpu.SemaphoreType.DMA,
        ],
    )
    return fn(tab, tab2, row, col)


_MLP_PBLK = 4096


def _mlp_body(gsum_ref, e_ref, se_ref, te_ref, w1e_ref, b1_ref,
              w2_ref, b2_ref, w3_ref, b3_ref, w4_ref, b4_ref,
              w5_ref, b5_ref, out_ref):
    ep = e_ref[...] * se_ref[...] + te_ref[...]
    h = gsum_ref[...] + b1_ref[...] + jnp.dot(
        ep, w1e_ref[...], preferred_element_type=jnp.float32)
    h = _lrelu(h)
    h = _lrelu(jnp.dot(h, w2_ref[...],
                       preferred_element_type=jnp.float32) + b2_ref[...])
    h = _lrelu(jnp.dot(h, w3_ref[...],
                       preferred_element_type=jnp.float32) + b3_ref[...])
    h = _lrelu(jnp.dot(h, w4_ref[...],
                       preferred_element_type=jnp.float32) + b4_ref[...])
    out_ref[...] = jnp.dot(h, w5_ref[...],
                           preferred_element_type=jnp.float32) + b5_ref[...]


def _run_mlp(gsum_p, e, se, te, w1e_bd, b1p, w2bd, b2p, w3bd, b3p,
             w4bd, b4p, w5bd, b5p):
    n2 = gsum_p.shape[0]
    const = lambda *_: (0, 0)
    return pl.pallas_call(
        _mlp_body,
        grid=((n2 + _MLP_PBLK - 1) // _MLP_PBLK,),
        in_specs=[
            pl.BlockSpec((_MLP_PBLK, 2 * H1), lambda i: (i, 0)),
            pl.BlockSpec((_MLP_PBLK, 2 * EDGE_IN), lambda i: (i, 0)),
            pl.BlockSpec((1, 2 * EDGE_IN), const),
            pl.BlockSpec((1, 2 * EDGE_IN), const),
            pl.BlockSpec((2 * EDGE_IN, 2 * H1), const),
            pl.BlockSpec((1, 2 * H1), const),
            pl.BlockSpec((2 * H1, 64), const),
            pl.BlockSpec((1, 64), const),
            pl.BlockSpec((64, 32), const),
            pl.BlockSpec((1, 32), const),
            pl.BlockSpec((32, 16), const),
            pl.BlockSpec((1, 16), const),
            pl.BlockSpec((16, 4), const),
            pl.BlockSpec((1, 4), const),
        ],
        out_specs=pl.BlockSpec((_MLP_PBLK, 4), lambda i: (i, 0)),
        out_shape=jax.ShapeDtypeStruct((n2, 4), jnp.float32),
    )(gsum_p, e, se, te, w1e_bd, b1p, w2bd, b2p, w3bd, b3p, w4bd, b4p,
      w5bd, b5p)


def kernel(x, edge_index, e, xbatch, bn_node_gamma, bn_node_beta,
           bn_edge_gamma, bn_edge_beta, W1, b1, W2, b2, W3, b3, W4, b4,
           W5, b5):
    x = x.reshape(-1, NODE_IN)
    e = e.reshape(-1, EDGE_IN)
    row = edge_index[0]
    col = edge_index[1]

    w1r_t = W1[:, :NODE_IN].T
    w1c_t = W1[:, NODE_IN:2 * NODE_IN].T
    w1e_t = W1[:, 2 * NODE_IN:].T
    gn = bn_node_gamma.reshape(1, NODE_IN)
    bn = bn_node_beta.reshape(1, NODE_IN)
    ge = bn_edge_gamma.reshape(1, EDGE_IN)
    be = bn_edge_beta.reshape(1, EDGE_IN)

    tab, tab2, b1_eff = _run_proj(x, gn, bn, w1r_t, w1c_t, b1.reshape(1, H1))
    gsum_p = _run_gather(tab, tab2, row, col)
    se, te = _run_estats(e, ge, be)

    def _bd(w):
        z = jnp.zeros_like(w)
        return jnp.concatenate(
            [jnp.concatenate([w, z], axis=1),
             jnp.concatenate([z, w], axis=1)], axis=0)

    def _pp(v):
        v = v.reshape(1, -1)
        return jnp.concatenate([v, v], axis=1)

    e_p = e.reshape(N_EDGES // 2, 2 * EDGE_IN)
    out_p = _run_mlp(
        gsum_p, e_p,
        jnp.concatenate([se, se], axis=1),
        jnp.concatenate([te, te], axis=1),
        _bd(w1e_t),
        jnp.concatenate([b1_eff, b1_eff], axis=1),
        _bd(W2.T), _pp(b2), _bd(W3.T), _pp(b3),
        _bd(W4.T), _pp(b4), _bd(W5.T), _pp(b5))
    return out_p.reshape(N_EDGES, 2)


# speedup vs baseline: 1.1935x; 1.1935x over previous
import functools

import jax
import jax.numpy as jnp
from jax import lax
from jax.experimental import pallas as pl
from jax.experimental.pallas import tpu as pltpu
from jax.experimental.pallas import tpu_sc as plsc

NODE_IN = 256
EDGE_IN = 16
LEAK = 0.1
N_NODES = 10000
N_EDGES = 160000
H1 = 64

_HIGH = lax.Precision.HIGHEST


def _lrelu(v):
    return jnp.where(v >= 0, v, LEAK * v)


_E_STEPS = 5
_E_BLK = N_EDGES // _E_STEPS


def _estats_body(e_ref, ge_ref, be_ref, se_ref, te_ref, acce_ref):
    i = pl.program_id(0)

    @pl.when(i == 0)
    def _init():
        acce_ref[...] = jnp.zeros_like(acce_ref)

    eb = e_ref[...]
    acce_ref[0:1, :] += jnp.sum(eb, axis=0, keepdims=True)
    acce_ref[1:2, :] += jnp.sum(eb * eb, axis=0, keepdims=True)

    @pl.when(i == _E_STEPS - 1)
    def _finish():
        mu_e = acce_ref[0:1, :] / N_EDGES
        var_e = acce_ref[1:2, :] / N_EDGES - mu_e * mu_e
        se = ge_ref[...] * lax.rsqrt(var_e + 1e-5)
        se_ref[...] = se
        te_ref[...] = be_ref[...] - mu_e * se


def _run_estats(e, ge, be):
    const = lambda *_: (0, 0)
    return pl.pallas_call(
        _estats_body,
        grid=(_E_STEPS,),
        in_specs=[
            pl.BlockSpec((_E_BLK, EDGE_IN), lambda i: (i, 0)),
            pl.BlockSpec((1, EDGE_IN), const),
            pl.BlockSpec((1, EDGE_IN), const),
        ],
        out_specs=(
            pl.BlockSpec((1, EDGE_IN), const),
            pl.BlockSpec((1, EDGE_IN), const),
        ),
        out_shape=(
            jax.ShapeDtypeStruct((1, EDGE_IN), jnp.float32),
            jax.ShapeDtypeStruct((1, EDGE_IN), jnp.float32),
        ),
        scratch_shapes=[pltpu.VMEM((2, EDGE_IN), jnp.float32)],
    )(e, ge, be)


def _proj_body(x_ref, gn_ref, bn_ref, w1r_ref, w1c_ref, b1_ref,
               t_ref, b1e_ref):
    x = x_ref[...]
    mu = jnp.mean(x, axis=0, keepdims=True)
    xm = x - mu
    var = jnp.mean(xm * xm, axis=0, keepdims=True)
    sx = gn_ref[...] * lax.rsqrt(var + 1e-5)
    tx = bn_ref[...] - mu * sx
    xbn = x * sx + tx
    w1r = w1r_ref[...]
    w1c = w1c_ref[...]
    xr = jnp.dot(xbn, w1r, preferred_element_type=jnp.float32)
    xc = jnp.dot(xbn, w1c, preferred_element_type=jnp.float32)
    t_ref[...] = jnp.concatenate([xr, xc], axis=1)
    b1e_ref[...] = (b1_ref[...]
                    + jnp.dot(tx, w1r, preferred_element_type=jnp.float32)
                    + jnp.dot(tx, w1c, preferred_element_type=jnp.float32))


def _run_proj(x, gn, bn, w1r_t, w1c_t, b1):
    return pl.pallas_call(
        _proj_body,
        out_shape=(
            jax.ShapeDtypeStruct((N_NODES, 2 * H1), jnp.float32),
            jax.ShapeDtypeStruct((1, H1), jnp.float32),
        ),
    )(x, gn, bn, w1r_t, w1c_t, b1)


_NW = 32
_GC = 128


def _make_gather_body(epw, nfull, rem):
    def _gather_body(tab_hbm, row_hbm, col_hbm, gsum_hbm,
                     idx_row, idx_col, br0, br1, bc0, bc1, ob0, ob1,
                     sem_r0, sem_r1, sem_c0, sem_c1, sem_o0, sem_o1):
        wid = lax.axis_index("s") * 2 + lax.axis_index("c")
        base0 = wid * epw
        pltpu.sync_copy(row_hbm.at[pl.ds(base0, epw)], idx_row)
        pltpu.sync_copy(col_hbm.at[pl.ds(base0, epw)], idx_col)

        brs = (br0, br1)
        bcs = (bc0, bc1)
        obs = (ob0, ob1)
        sem_rs = (sem_r0, sem_r1)
        sem_cs = (sem_c0, sem_c1)
        sem_os = (sem_o0, sem_o1)

        def fire(j, b):
            sl = pl.ds(j * _GC, _GC)
            pltpu.async_copy(tab_hbm.at[idx_row.at[sl]], brs[b], sem_rs[b])
            pltpu.async_copy(tab_hbm.at[idx_col.at[sl]], bcs[b], sem_cs[b])

        def wait_gather(b):
            sl = pl.ds(0, _GC)
            pltpu.make_async_copy(tab_hbm.at[idx_row.at[sl]], brs[b],
                                  sem_rs[b]).wait()
            pltpu.make_async_copy(tab_hbm.at[idx_col.at[sl]], bcs[b],
                                  sem_cs[b]).wait()

        def wait_out(b):
            pltpu.make_async_copy(obs[b],
                                  gsum_hbm.at[pl.ds(base0, _GC)],
                                  sem_os[b]).wait()

        def process(j, b):
            def row_add(r, carry):
                for q in range(H1 // 16):
                    obs[b][r, pl.ds(q * 16, 16)] = (
                        brs[b][r, pl.ds(q * 16, 16)]
                        + bcs[b][r, pl.ds(H1 + q * 16, 16)])
                return carry
            lax.fori_loop(0, _GC, row_add, 0)
            pltpu.async_copy(obs[b],
                             gsum_hbm.at[pl.ds(base0 + j * _GC, _GC)],
                             sem_os[b])

        fire(0, 0)
        fire(1, 1)

        def outer(j0, carry):
            for b in range(2):
                j = 2 * j0 + b
                wait_gather(b)

                @pl.when(j >= 2)
                def _():
                    wait_out(b)
                process(j, b)

                @pl.when(j + 2 < nfull)
                def _():
                    fire(j + 2, b)
            return carry

        lax.fori_loop(0, (nfull - 1) // 2, outer, 0)

        j_last = nfull - 1
        wait_gather(0)
        wait_out(0)
        process(j_last, 0)

        sl = pl.ds(nfull * _GC, rem)
        rem_r = br1.at[pl.ds(0, rem)]
        rem_c = bc1.at[pl.ds(0, rem)]
        wait_out(1)
        pltpu.async_copy(tab_hbm.at[idx_row.at[sl]], rem_r, sem_r1)
        pltpu.async_copy(tab_hbm.at[idx_col.at[sl]], rem_c, sem_c1)
        pltpu.make_async_copy(tab_hbm.at[idx_row.at[sl]], rem_r,
                              sem_r1).wait()
        pltpu.make_async_copy(tab_hbm.at[idx_col.at[sl]], rem_c,
                              sem_c1).wait()

        def rem_add(r, carry):
            for q in range(H1 // 16):
                ob1[r, pl.ds(q * 16, 16)] = (
                    br1[r, pl.ds(q * 16, 16)]
                    + bc1[r, pl.ds(H1 + q * 16, 16)])
            return carry
        lax.fori_loop(0, rem, rem_add, 0)
        pltpu.sync_copy(ob1.at[pl.ds(0, rem)],
                        gsum_hbm.at[pl.ds(base0 + nfull * _GC, rem)])
        wait_out(0)

    return _gather_body


def _run_gather(tab, row, col):
    n = row.shape[0]
    epw = n // _NW
    nfull = epw // _GC
    rem = epw - nfull * _GC
    assert n % (_NW * 8) == 0 and nfull % 2 == 1 and rem % 2 == 0 and rem > 0
    mesh = plsc.VectorSubcoreMesh(core_axis_name="c", subcore_axis_name="s",
                                  num_cores=2, num_subcores=16)
    fn = pl.kernel(
        _make_gather_body(epw, nfull, rem),
        out_type=jax.ShapeDtypeStruct((n, H1), jnp.float32),
        mesh=mesh,
        scratch_types=[
            pltpu.VMEM((epw,), jnp.int32),
            pltpu.VMEM((epw,), jnp.int32),
            pltpu.VMEM((_GC, 2 * H1), jnp.float32),
            pltpu.VMEM((_GC, 2 * H1), jnp.float32),
            pltpu.VMEM((_GC, 2 * H1), jnp.float32),
            pltpu.VMEM((_GC, 2 * H1), jnp.float32),
            pltpu.VMEM((_GC, H1), jnp.float32),
            pltpu.VMEM((_GC, H1), jnp.float32),
            pltpu.SemaphoreType.DMA,
            pltpu.SemaphoreType.DMA,
            pltpu.SemaphoreType.DMA,
            pltpu.SemaphoreType.DMA,
            pltpu.SemaphoreType.DMA,
            pltpu.SemaphoreType.DMA,
        ],
    )
    return fn(tab, row, col)


_MLP_BLK = 8192


def _mlp_body(gsum_ref, e_ref, se_ref, te_ref, w1e_ref, b1_ref,
              w2_ref, b2_ref, w3_ref, b3_ref, w4_ref, b4_ref,
              w5_ref, b5_ref, out_ref):
    ebn = e_ref[...] * se_ref[...] + te_ref[...]
    h = gsum_ref[...] + b1_ref[...] + jnp.dot(
        ebn, w1e_ref[...], preferred_element_type=jnp.float32)
    h = _lrelu(h)
    h = _lrelu(jnp.dot(h, w2_ref[...],
                       preferred_element_type=jnp.float32) + b2_ref[...])
    h = _lrelu(jnp.dot(h, w3_ref[...],
                       preferred_element_type=jnp.float32) + b3_ref[...])
    h = _lrelu(jnp.dot(h, w4_ref[...],
                       preferred_element_type=jnp.float32) + b4_ref[...])
    out_ref[...] = jnp.dot(h, w5_ref[...],
                           preferred_element_type=jnp.float32) + b5_ref[...]


def _run_mlp(gsum, e, se, te, w1e_t, b1_eff, w2_t, b2, w3_t, b3, w4_t, b4,
             w5_t, b5):
    n = gsum.shape[0]
    const = lambda *_: (0, 0)
    return pl.pallas_call(
        _mlp_body,
        grid=((n + _MLP_BLK - 1) // _MLP_BLK,),
        in_specs=[
            pl.BlockSpec((_MLP_BLK, H1), lambda i: (i, 0)),
            pl.BlockSpec((_MLP_BLK, EDGE_IN), lambda i: (i, 0)),
            pl.BlockSpec((1, EDGE_IN), const),
            pl.BlockSpec((1, EDGE_IN), const),
            pl.BlockSpec((EDGE_IN, H1), const),
            pl.BlockSpec((1, H1), const),
            pl.BlockSpec((H1, 32), const),
            pl.BlockSpec((1, 32), const),
            pl.BlockSpec((32, 16), const),
            pl.BlockSpec((1, 16), const),
            pl.BlockSpec((16, 8), const),
            pl.BlockSpec((1, 8), const),
            pl.BlockSpec((8, 2), const),
            pl.BlockSpec((1, 2), const),
        ],
        out_specs=pl.BlockSpec((_MLP_BLK, 2), lambda i: (i, 0)),
        out_shape=jax.ShapeDtypeStruct((n, 2), jnp.float32),
    )(gsum, e, se, te, w1e_t, b1_eff, w2_t, b2, w3_t, b3, w4_t, b4,
      w5_t, b5)


def kernel(x, edge_index, e, xbatch, bn_node_gamma, bn_node_beta,
           bn_edge_gamma, bn_edge_beta, W1, b1, W2, b2, W3, b3, W4, b4,
           W5, b5):
    x = x.reshape(-1, NODE_IN)
    e = e.reshape(-1, EDGE_IN)
    row = edge_index[0]
    col = edge_index[1]

    w1r_t = W1[:, :NODE_IN].T
    w1c_t = W1[:, NODE_IN:2 * NODE_IN].T
    w1e_t = W1[:, 2 * NODE_IN:].T
    gn = bn_node_gamma.reshape(1, NODE_IN)
    bn = bn_node_beta.reshape(1, NODE_IN)
    ge = bn_edge_gamma.reshape(1, EDGE_IN)
    be = bn_edge_beta.reshape(1, EDGE_IN)

    tab, b1_eff = _run_proj(x, gn, bn, w1r_t, w1c_t, b1.reshape(1, H1))
    gsum = _run_gather(tab, row, col)
    se, te = _run_estats(e, ge, be)
    return _run_mlp(
        gsum, e, se, te, w1e_t, b1_eff,
        W2.T, b2.reshape(1, -1), W3.T, b3.reshape(1, -1),
        W4.T, b4.reshape(1, -1), W5.T, b5.reshape(1, -1))
